# Initial kernel scaffold; baseline (speedup 1.0000x reference)
import jax
import jax.numpy as jnp
from jax.experimental import pallas as pl


def kernel(h, q, W1, W2, edge_index):
    raise NotImplementedError("write your pallas kernel here")


# speedup vs baseline: 14.5574x; 14.5574x over previous
import dataclasses
import functools

import jax
import jax.numpy as jnp
from jax import lax
from jax.experimental import pallas as pl
from jax.experimental.pallas import tpu as pltpu
from jax.experimental.pallas import tpu_sc as plsc

N = 10000
N_PAD = 10240
D = 128
HD = 64
QD = 128
SLOPE = 0.2
E = 320000
NC = 2
NS = 16
NW = NC * NS
EPS = E // NS
CH = 80
NCH = EPS // CH
DCH = NCH // NC
ROWS_PER_SUB = N_PAD // NS
ZROWS = 128
BLK = 1024


def _payload_block(h_blk, q_blk, w_vec):
    wh = w_vec[:, :D]
    wq = w_vec[:, D:]
    z = (jnp.sum(h_blk * wh, axis=1, keepdims=True)
         + jnp.sum(q_blk * wq, axis=1, keepdims=True))
    a = jnp.where(z > 0, z, SLOPE * z)
    w = jnp.exp(a)
    g = h_blk * w
    return jnp.stack([g[:, :HD], g[:, HD:]], axis=0), w[:, 0]


def _combine_block(num_ref, den_ref, h_prev):
    num = jnp.concatenate([num_ref[0], num_ref[1]], axis=1)
    den = jnp.sum(den_ref[...], axis=(0, 1))[:, None]
    keep = den > 0.0
    return jnp.where(keep, num / jnp.where(keep, den, 1.0), h_prev)


def _prologue_body(h_ref, q_ref, w_ref, g_ref, wout_ref):
    g, w = _payload_block(h_ref[...], q_ref[...], w_ref[...])
    g_ref[...] = g
    wout_ref[...] = w


def _mid_body(num_ref, den_ref, h_ref, q_ref, w_ref, h1_ref, g2_ref, w2_ref):
    h1 = _combine_block(num_ref, den_ref, h_ref[...])
    h1_ref[...] = h1
    g2, w2 = _payload_block(h1, q_ref[...], w_ref[...])
    g2_ref[...] = g2
    w2_ref[...] = w2


def _final_body(num_ref, den_ref, h1_ref, out_ref):
    out_ref[...] = _combine_block(num_ref, den_ref, h1_ref[...])


def _tc_prologue(h_p, q_p, w_vec):
    return pl.pallas_call(
        _prologue_body,
        grid=(N_PAD // BLK,),
        in_specs=[
            pl.BlockSpec((BLK, D), lambda i: (i, 0)),
            pl.BlockSpec((BLK, QD), lambda i: (i, 0)),
            pl.BlockSpec((1, D + QD), lambda i: (0, 0)),
        ],
        out_specs=[
            pl.BlockSpec((NC, BLK, HD), lambda i: (0, i, 0)),
            pl.BlockSpec((BLK,), lambda i: (i,)),
        ],
        out_shape=[
            jax.ShapeDtypeStruct((NC, N_PAD, HD), jnp.float32),
            jax.ShapeDtypeStruct((N_PAD,), jnp.float32),
        ],
    )(h_p, q_p, w_vec)


def _tc_mid(num, den, h_p, q_p, w_vec):
    return pl.pallas_call(
        _mid_body,
        grid=(N_PAD // BLK,),
        in_specs=[
            pl.BlockSpec((NC, BLK, HD), lambda i: (0, i, 0)),
            pl.BlockSpec((NC, NS, BLK), lambda i: (0, 0, i)),
            pl.BlockSpec((BLK, D), lambda i: (i, 0)),
            pl.BlockSpec((BLK, QD), lambda i: (i, 0)),
            pl.BlockSpec((1, D + QD), lambda i: (0, 0)),
        ],
        out_specs=[
            pl.BlockSpec((BLK, D), lambda i: (i, 0)),
            pl.BlockSpec((NC, BLK, HD), lambda i: (0, i, 0)),
            pl.BlockSpec((BLK,), lambda i: (i,)),
        ],
        out_shape=[
            jax.ShapeDtypeStruct((N_PAD, D), jnp.float32),
            jax.ShapeDtypeStruct((NC, N_PAD, HD), jnp.float32),
            jax.ShapeDtypeStruct((N_PAD,), jnp.float32),
        ],
    )(num, den, h_p, q_p, w_vec)


def _tc_final(num, den, h1):
    return pl.pallas_call(
        _final_body,
        grid=(N_PAD // BLK,),
        in_specs=[
            pl.BlockSpec((NC, BLK, HD), lambda i: (0, i, 0)),
            pl.BlockSpec((NC, NS, BLK), lambda i: (0, 0, i)),
            pl.BlockSpec((BLK, D), lambda i: (i, 0)),
        ],
        out_specs=pl.BlockSpec((BLK, D), lambda i: (i, 0)),
        out_shape=jax.ShapeDtypeStruct((N_PAD, D), jnp.float32),
    )(num, den, h1)


def _sc_segment_sum(g, w, src3, dst3):
    mesh = plsc.VectorSubcoreMesh(core_axis_name="c", subcore_axis_name="s")
    cparams = pltpu.CompilerParams()
    flds = pltpu.CompilerParams.__dataclass_fields__
    if "needs_layout_passes" in flds:
        cparams = dataclasses.replace(cparams, needs_layout_passes=False)
    if "use_tc_tiling_on_sc" in flds:
        cparams = dataclasses.replace(cparams, use_tc_tiling_on_sc=False)

    @functools.partial(
        pl.kernel,
        compiler_params=cparams,
        out_type=[
            jax.ShapeDtypeStruct((NC, N_PAD, HD), jnp.float32),
            jax.ShapeDtypeStruct((NC, NS, N_PAD), jnp.float32),
        ],
        mesh=mesh,
        scratch_types=[
            pltpu.VMEM((NCH, CH), jnp.int32),
            pltpu.VMEM((NCH, CH), jnp.int32),
            pltpu.VMEM((CH, HD), jnp.float32),
            pltpu.VMEM((ZROWS, HD), jnp.float32),
            pltpu.VMEM((N_PAD,), jnp.float32),
            pltpu.VMEM((N_PAD,), jnp.float32),
            pltpu.VMEM_SHARED((N_PAD, HD), jnp.float32),
            pltpu.SemaphoreType.DMA,
        ],
    )
    def sc_kernel(g_hbm, w_hbm, src_hbm, dst_hbm, num_hbm, den_hbm,
                  src_v, dst_v, rows_v, zbuf, w_v, den_v, acc_sh, sem):
        cid = lax.axis_index("c")
        sid = lax.axis_index("s")

        zero = jnp.zeros((16,), jnp.float32)

        @pl.loop(0, ZROWS)
        def _(r):
            @pl.loop(0, HD, step=16)
            def _(col):
                zbuf[r, pl.ds(col, 16)] = zero

        @pl.loop(0, N_PAD, step=16)
        def _(i):
            den_v[pl.ds(i, 16)] = zero

        @pl.loop(0, ROWS_PER_SUB, step=ZROWS)
        def _(k):
            pltpu.sync_copy(zbuf, acc_sh.at[pl.ds(sid * ROWS_PER_SUB + k, ZROWS)])

        plsc.subcore_barrier()

        pltpu.sync_copy(src_hbm.at[sid], src_v)
        pltpu.sync_copy(dst_hbm.at[sid], dst_v)
        pltpu.sync_copy(w_hbm, w_v)

        den_lo = cid * DCH

        @pl.loop(0, NCH)
        def _(j):
            cp = pltpu.async_copy(g_hbm.at[cid].at[src_v.at[j]], rows_v, sem)

            @pl.when(jnp.logical_and(j >= den_lo, j < den_lo + DCH))
            def _():
                @pl.loop(0, CH, step=16)
                def _(c):
                    srcv = src_v[j, pl.ds(c, 16)]
                    dstv = dst_v[j, pl.ds(c, 16)]
                    wv = plsc.load_gather(w_v, [srcv])
                    plsc.addupdate_scatter(den_v, [dstv], wv)

            cp.wait()
            pltpu.sync_copy(rows_v, acc_sh.at[dst_v.at[j]], add=True)

        plsc.subcore_barrier()

        pltpu.sync_copy(
            acc_sh.at[pl.ds(sid * ROWS_PER_SUB, ROWS_PER_SUB)],
            num_hbm.at[cid].at[pl.ds(sid * ROWS_PER_SUB, ROWS_PER_SUB)],
        )
        pltpu.sync_copy(den_v, den_hbm.at[cid].at[sid])

    return sc_kernel(g, w, src3, dst3)


def kernel(h, q, W1, W2, edge_index):
    h_p = jnp.pad(h, ((0, N_PAD - N), (0, 0)))
    q_p = jnp.pad(q, ((0, N_PAD - N), (0, 0)))
    w1 = W1[:, 0][None, :]
    w2 = W2[:, 0][None, :]
    src3 = edge_index[0].reshape(NS, NCH, CH)
    dst3 = edge_index[1].reshape(NS, NCH, CH)

    g1, wn1 = _tc_prologue(h_p, q_p, w1)
    num1, den1 = _sc_segment_sum(g1, wn1, src3, dst3)
    h1, g2, wn2 = _tc_mid(num1, den1, h_p, q_p, w2)
    num2, den2 = _sc_segment_sum(g2, wn2, src3, dst3)
    h2 = _tc_final(num2, den2, h1)
    return h2[:N]
